# baseline (speedup 1.0000x reference)
import functools

import jax
import jax.numpy as jnp
from jax import lax
from jax.experimental import pallas as pl
from jax.experimental.pallas import tpu as pltpu
from jax.experimental.pallas import tpu_sc as plsc

_E = 8
_D = 1024
_N = 4096
_K = 2
_A = _N * _K
_TG = 256
_G = _A // _TG + _E
_S = _G * _TG
_CT = 512
_TT = 512

_NW = 32
_RPW = _A // _NW
_CH = 128
_NCH = _RPW // _CH


def _router_body(x_ref, rk_ref, rb_ref, dest_ref, gates_ref, tmeta_ref):
    x = x_ref[...]
    logits = jnp.dot(x, rk_ref[...], preferred_element_type=jnp.float32)
    logits = logits + rb_ref[...]
    m = jnp.max(logits, axis=-1, keepdims=True)
    ex = jnp.exp(logits - m)
    probs = ex / jnp.sum(ex, axis=-1, keepdims=True)

    lane = lax.broadcasted_iota(jnp.int32, probs.shape, 1)
    p1 = jnp.max(probs, axis=-1, keepdims=True)
    i1 = jnp.min(jnp.where(probs == p1, lane, _E), axis=-1, keepdims=True)
    probs2 = jnp.where(lane == i1, -jnp.inf, probs)
    p2 = jnp.max(probs2, axis=-1, keepdims=True)
    i2 = jnp.min(jnp.where(probs2 == p2, lane, _E), axis=-1, keepdims=True)
    denom = p1 + p2
    gates_ref[...] = jnp.concatenate([p1 / denom, p2 / denom], axis=1)

    oh0 = (lane == i1).astype(jnp.float32)
    oh1 = (lane == i2).astype(jnp.float32)

    r = lax.broadcasted_iota(jnp.int32, (_CT, _CT), 0)
    c = lax.broadcasted_iota(jnp.int32, (_CT, _CT), 1)
    tril = (c < r).astype(jnp.bfloat16)
    carry0 = jnp.zeros((1, _E), jnp.float32)
    carry1 = jnp.zeros((1, _E), jnp.float32)
    r0_parts, r1_parts = [], []
    for t in range(_N // _CT):
        s0 = oh0[t * _CT:(t + 1) * _CT]
        s1 = oh1[t * _CT:(t + 1) * _CT]
        r0_parts.append(jnp.dot(tril, s0.astype(jnp.bfloat16),
                                preferred_element_type=jnp.float32) + carry0)
        r1_parts.append(jnp.dot(tril, s1.astype(jnp.bfloat16),
                                preferred_element_type=jnp.float32) + carry1)
        carry0 = carry0 + jnp.sum(s0, axis=0, keepdims=True)
        carry1 = carry1 + jnp.sum(s1, axis=0, keepdims=True)
    ranks0 = jnp.concatenate(r0_parts, axis=0)
    ranks1 = jnp.concatenate(r1_parts, axis=0) + carry0
    counts = carry0 + carry1

    ptiles = jnp.floor((counts + (_TG - 1.0)) * (1.0 / _TG))
    rr = lax.broadcasted_iota(jnp.int32, (_E, _E), 0)
    cc = lax.broadcasted_iota(jnp.int32, (_E, _E), 1)
    triu = (rr < cc).astype(jnp.bfloat16)
    tstart = jnp.dot(ptiles.astype(jnp.bfloat16), triu,
                     preferred_element_type=jnp.float32)
    toff = tstart * float(_TG)

    d0 = jnp.sum(oh0 * (toff + ranks0), axis=1, keepdims=True)
    d1 = jnp.sum(oh1 * (toff + ranks1), axis=1, keepdims=True)
    dest_ref[...] = jnp.concatenate([d0, d1], axis=1).astype(jnp.int32)

    gi = lax.broadcasted_iota(jnp.int32, (64, _E), 0).astype(jnp.float32)
    te = jnp.sum((gi >= tstart).astype(jnp.float32), axis=1, keepdims=True)
    tmeta_ref[...] = (te - 1.0).astype(jnp.int32)


def _route(xbf, rkbf, rb):
    return pl.pallas_call(
        _router_body,
        in_specs=[
            pl.BlockSpec((_N, _D), lambda: (0, 0)),
            pl.BlockSpec((_D, _E), lambda: (0, 0)),
            pl.BlockSpec((1, _E), lambda: (0, 0)),
        ],
        out_specs=[
            pl.BlockSpec((_N, _K), lambda: (0, 0)),
            pl.BlockSpec((_N, _K), lambda: (0, 0)),
            pl.BlockSpec((64, 1), lambda: (0, 0)),
        ],
        out_shape=[
            jax.ShapeDtypeStruct((_N, _K), jnp.int32),
            jax.ShapeDtypeStruct((_N, _K), jnp.float32),
            jax.ShapeDtypeStruct((64, 1), jnp.int32),
        ],
    )(xbf, rkbf, rb)


@functools.lru_cache(maxsize=None)
def _sc_kernels():
    mesh = plsc.VectorSubcoreMesh(core_axis_name="c", subcore_axis_name="s")

    @functools.partial(
        pl.kernel, mesh=mesh,
        out_type=jax.ShapeDtypeStruct((_S, _D // 2), jnp.int32),
        scratch_types=[
            pltpu.VMEM((_NCH, _CH), jnp.int32),
            pltpu.VMEM((_CH, _D // 2), jnp.int32),
            pltpu.SemaphoreType.DMA,
        ],
    )
    def sc_scatter(x_hbm, idx_hbm, xs_hbm, idx_v, rows_v, sem):
        wid = lax.axis_index("s") * 2 + lax.axis_index("c")
        base = wid * _RPW
        pltpu.sync_copy(idx_hbm.at[pl.ds(wid * _NCH, _NCH)], idx_v)
        for j in range(_NCH):
            src_tok = (base + j * _CH) % _N
            pltpu.sync_copy(x_hbm.at[pl.ds(src_tok, _CH)], rows_v)
            pltpu.async_copy(rows_v, xs_hbm.at[idx_v.at[j]], sem).wait()

    @functools.partial(
        pl.kernel, mesh=mesh,
        out_type=jax.ShapeDtypeStruct((_A, _D // 2), jnp.int32),
        scratch_types=[
            pltpu.VMEM((_NCH, _CH), jnp.int32),
            pltpu.VMEM((_CH, _D // 2), jnp.int32),
            pltpu.SemaphoreType.DMA,
        ],
    )
    def sc_gather(ys_hbm, idx_hbm, rows_hbm, idx_v, rows_v, sem):
        wid = lax.axis_index("s") * 2 + lax.axis_index("c")
        base = wid * _RPW
        pltpu.sync_copy(idx_hbm.at[pl.ds(wid * _NCH, _NCH)], idx_v)
        for j in range(_NCH):
            pltpu.async_copy(ys_hbm.at[idx_v.at[j]], rows_v, sem).wait()
            pltpu.sync_copy(rows_v, rows_hbm.at[pl.ds(base + j * _CH, _CH)])

    return sc_scatter, sc_gather


def _expert_body(te_ref, xs_ref, ek_ref, eb_ref, ys_ref):
    h = jnp.dot(xs_ref[...], ek_ref[0], preferred_element_type=jnp.float32)
    h = h + eb_ref[0]
    ys_ref[...] = jax.nn.gelu(h).astype(jnp.bfloat16)


def _expert_gemm(tile_expert, xs, ekbf, eb):
    grid_spec = pltpu.PrefetchScalarGridSpec(
        num_scalar_prefetch=1,
        grid=(_G,),
        in_specs=[
            pl.BlockSpec((_TG, _D), lambda g, te: (g, 0)),
            pl.BlockSpec((1, _D, _D), lambda g, te: (te[g], 0, 0)),
            pl.BlockSpec((1, 1, _D), lambda g, te: (te[g], 0, 0)),
        ],
        out_specs=pl.BlockSpec((_TG, _D), lambda g, te: (g, 0)),
    )
    return pl.pallas_call(
        _expert_body,
        grid_spec=grid_spec,
        out_shape=jax.ShapeDtypeStruct((_S, _D), jnp.bfloat16),
        compiler_params=pltpu.CompilerParams(
            dimension_semantics=("arbitrary",),
        ),
    )(tile_expert, xs, ekbf, eb)


def _combine_body(r0_ref, r1_ref, gates_ref, wo_ref, ob_ref, out_ref):
    g = gates_ref[...]
    comb = (r0_ref[...].astype(jnp.float32) * g[:, 0:1]
            + r1_ref[...].astype(jnp.float32) * g[:, 1:2])
    out = jnp.dot(comb.astype(jnp.bfloat16), wo_ref[...],
                  preferred_element_type=jnp.float32)
    out_ref[...] = out + ob_ref[...]


def _combine(rows, gates, wobf, ob):
    grid = (_N // _TT,)
    return pl.pallas_call(
        _combine_body,
        grid=grid,
        in_specs=[
            pl.BlockSpec((_TT, _D), lambda i: (i, 0)),
            pl.BlockSpec((_TT, _D), lambda i: (i + _N // _TT, 0)),
            pl.BlockSpec((_TT, _K), lambda i: (i, 0)),
            pl.BlockSpec((_D, _D), lambda i: (0, 0)),
            pl.BlockSpec((1, _D), lambda i: (0, 0)),
        ],
        out_specs=pl.BlockSpec((_TT, _D), lambda i: (i, 0)),
        out_shape=jax.ShapeDtypeStruct((_N, _D), jnp.float32),
        compiler_params=pltpu.CompilerParams(
            dimension_semantics=("arbitrary",),
        ),
    )(rows, rows, gates, wobf, ob)


@jax.jit
def kernel(x, router_kernel, router_bias, expert_kernels, expert_biases,
           out_kernel, out_bias):
    b, s, d = x.shape
    xbf = x.reshape(b * s, d).astype(jnp.bfloat16)
    rkbf = router_kernel.astype(jnp.bfloat16)
    ekbf = expert_kernels.astype(jnp.bfloat16)
    wobf = out_kernel.astype(jnp.bfloat16)
    rb = router_bias.reshape(1, _E)
    ob = out_bias.reshape(1, _D)

    dest, gates, tmeta = _route(xbf, rkbf, rb)
    idx = dest.T.reshape(_A // _CH, _CH)
    tile_expert = tmeta.reshape(64)[:_G]

    sc_scatter, sc_gather = _sc_kernels()
    x_i32 = lax.bitcast_convert_type(xbf.reshape(_N, _D // 2, 2), jnp.int32)
    xs_i32 = sc_scatter(x_i32, idx)
    xs = lax.bitcast_convert_type(xs_i32, jnp.bfloat16).reshape(_S, _D)
    ys = _expert_gemm(tile_expert, xs, ekbf, expert_biases.reshape(_E, 1, _D))
    ys_i32 = lax.bitcast_convert_type(ys.reshape(_S, _D // 2, 2), jnp.int32)
    rows_i32 = sc_gather(ys_i32, idx)
    rows = lax.bitcast_convert_type(rows_i32, jnp.bfloat16).reshape(_A, _D)
    out = _combine(rows, gates, wobf, ob)
    return out.reshape(b, s, d)


# speedup vs baseline: 4.3372x; 4.3372x over previous
import functools

import jax
import jax.numpy as jnp
from jax import lax
from jax.experimental import pallas as pl
from jax.experimental.pallas import tpu as pltpu
from jax.experimental.pallas import tpu_sc as plsc

_E = 8
_D = 1024
_N = 4096
_K = 2
_A = _N * _K
_TG = 256
_G = _A // _TG + _E
_S = _G * _TG
_CT = 512
_TT = 512

_NW = 32
_RPW = _A // _NW
_CH = 64
_NCH = _RPW // _CH


def _router_body(x_ref, rk_ref, rb_ref, dest_ref, gates_ref, tmeta_ref):
    x = x_ref[...]
    logits = jnp.dot(x, rk_ref[...], preferred_element_type=jnp.float32)
    logits = logits + rb_ref[...]
    m = jnp.max(logits, axis=-1, keepdims=True)
    ex = jnp.exp(logits - m)
    probs = ex / jnp.sum(ex, axis=-1, keepdims=True)

    lane = lax.broadcasted_iota(jnp.int32, probs.shape, 1)
    p1 = jnp.max(probs, axis=-1, keepdims=True)
    i1 = jnp.min(jnp.where(probs == p1, lane, _E), axis=-1, keepdims=True)
    probs2 = jnp.where(lane == i1, -jnp.inf, probs)
    p2 = jnp.max(probs2, axis=-1, keepdims=True)
    i2 = jnp.min(jnp.where(probs2 == p2, lane, _E), axis=-1, keepdims=True)
    denom = p1 + p2
    gates_ref[...] = jnp.concatenate([p1 / denom, p2 / denom], axis=1)

    oh0 = (lane == i1).astype(jnp.float32)
    oh1 = (lane == i2).astype(jnp.float32)

    r = lax.broadcasted_iota(jnp.int32, (_CT, _CT), 0)
    c = lax.broadcasted_iota(jnp.int32, (_CT, _CT), 1)
    tril = (c < r).astype(jnp.bfloat16)
    carry0 = jnp.zeros((1, _E), jnp.float32)
    carry1 = jnp.zeros((1, _E), jnp.float32)
    r0_parts, r1_parts = [], []
    for t in range(_N // _CT):
        s0 = oh0[t * _CT:(t + 1) * _CT]
        s1 = oh1[t * _CT:(t + 1) * _CT]
        r0_parts.append(jnp.dot(tril, s0.astype(jnp.bfloat16),
                                preferred_element_type=jnp.float32) + carry0)
        r1_parts.append(jnp.dot(tril, s1.astype(jnp.bfloat16),
                                preferred_element_type=jnp.float32) + carry1)
        carry0 = carry0 + jnp.sum(s0, axis=0, keepdims=True)
        carry1 = carry1 + jnp.sum(s1, axis=0, keepdims=True)
    ranks0 = jnp.concatenate(r0_parts, axis=0)
    ranks1 = jnp.concatenate(r1_parts, axis=0) + carry0
    counts = carry0 + carry1

    ptiles = jnp.floor((counts + (_TG - 1.0)) * (1.0 / _TG))
    rr = lax.broadcasted_iota(jnp.int32, (_E, _E), 0)
    cc = lax.broadcasted_iota(jnp.int32, (_E, _E), 1)
    triu = (rr < cc).astype(jnp.bfloat16)
    tstart = jnp.dot(ptiles.astype(jnp.bfloat16), triu,
                     preferred_element_type=jnp.float32)
    toff = tstart * float(_TG)

    d0 = jnp.sum(oh0 * (toff + ranks0), axis=1, keepdims=True)
    d1 = jnp.sum(oh1 * (toff + ranks1), axis=1, keepdims=True)
    dest_ref[...] = jnp.concatenate([d0, d1], axis=1).astype(jnp.int32)

    gi = lax.broadcasted_iota(jnp.int32, (64, _E), 0).astype(jnp.float32)
    te = jnp.sum((gi >= tstart).astype(jnp.float32), axis=1, keepdims=True)
    tmeta_ref[...] = (te - 1.0).astype(jnp.int32)


def _route(xbf, rkbf, rb):
    return pl.pallas_call(
        _router_body,
        in_specs=[
            pl.BlockSpec((_N, _D), lambda: (0, 0)),
            pl.BlockSpec((_D, _E), lambda: (0, 0)),
            pl.BlockSpec((1, _E), lambda: (0, 0)),
        ],
        out_specs=[
            pl.BlockSpec((_N, _K), lambda: (0, 0)),
            pl.BlockSpec((_N, _K), lambda: (0, 0)),
            pl.BlockSpec((64, 1), lambda: (0, 0)),
        ],
        out_shape=[
            jax.ShapeDtypeStruct((_N, _K), jnp.int32),
            jax.ShapeDtypeStruct((_N, _K), jnp.float32),
            jax.ShapeDtypeStruct((64, 1), jnp.int32),
        ],
    )(xbf, rkbf, rb)


@functools.lru_cache(maxsize=None)
def _sc_kernels():
    mesh = plsc.VectorSubcoreMesh(core_axis_name="c", subcore_axis_name="s")
    cp = pltpu.CompilerParams(use_tc_tiling_on_sc=True)

    @functools.partial(
        pl.kernel, mesh=mesh, compiler_params=cp,
        out_type=jax.ShapeDtypeStruct((_S, _D), jnp.float32),
        scratch_types=[
            pltpu.VMEM((_NCH, _CH), jnp.int32),
            pltpu.VMEM((_CH, _D), jnp.float32),
            pltpu.SemaphoreType.DMA,
        ],
    )
    def sc_scatter(x_hbm, idx_hbm, xs_hbm, idx_v, rows_v, sem):
        wid = lax.axis_index("s") * 2 + lax.axis_index("c")
        base = wid * _RPW
        pltpu.sync_copy(idx_hbm.at[wid], idx_v)
        for j in range(_NCH):
            src_tok = (base + j * _CH) % _N
            pltpu.sync_copy(x_hbm.at[pl.ds(src_tok, _CH)], rows_v)
            pltpu.async_copy(rows_v, xs_hbm.at[idx_v.at[j]], sem).wait()

    @functools.partial(
        pl.kernel, mesh=mesh, compiler_params=cp,
        out_type=jax.ShapeDtypeStruct((_A, _D), jnp.float32),
        scratch_types=[
            pltpu.VMEM((_NCH, _CH), jnp.int32),
            pltpu.VMEM((_CH, _D), jnp.float32),
            pltpu.SemaphoreType.DMA,
        ],
    )
    def sc_gather(ys_hbm, idx_hbm, rows_hbm, idx_v, rows_v, sem):
        wid = lax.axis_index("s") * 2 + lax.axis_index("c")
        base = wid * _RPW
        pltpu.sync_copy(idx_hbm.at[wid], idx_v)
        for j in range(_NCH):
            pltpu.async_copy(ys_hbm.at[idx_v.at[j]], rows_v, sem).wait()
            pltpu.sync_copy(rows_v, rows_hbm.at[pl.ds(base + j * _CH, _CH)])

    return sc_scatter, sc_gather


def _expert_body(te_ref, xs_ref, ek_ref, eb_ref, ys_ref):
    h = jnp.dot(xs_ref[...].astype(jnp.bfloat16), ek_ref[0],
                preferred_element_type=jnp.float32)
    h = h + eb_ref[0]
    ys_ref[...] = jax.nn.gelu(h)


def _expert_gemm(tile_expert, xs, ekbf, eb):
    grid_spec = pltpu.PrefetchScalarGridSpec(
        num_scalar_prefetch=1,
        grid=(_G,),
        in_specs=[
            pl.BlockSpec((_TG, _D), lambda g, te: (g, 0)),
            pl.BlockSpec((1, _D, _D), lambda g, te: (te[g], 0, 0)),
            pl.BlockSpec((1, 1, _D), lambda g, te: (te[g], 0, 0)),
        ],
        out_specs=pl.BlockSpec((_TG, _D), lambda g, te: (g, 0)),
    )
    return pl.pallas_call(
        _expert_body,
        grid_spec=grid_spec,
        out_shape=jax.ShapeDtypeStruct((_S, _D), jnp.float32),
        compiler_params=pltpu.CompilerParams(
            dimension_semantics=("arbitrary",),
        ),
    )(tile_expert, xs, ekbf, eb)


def _combine_body(r0_ref, r1_ref, gates_ref, wo_ref, ob_ref, out_ref):
    g = gates_ref[...]
    comb = r0_ref[...] * g[:, 0:1] + r1_ref[...] * g[:, 1:2]
    out = jnp.dot(comb.astype(jnp.bfloat16), wo_ref[...],
                  preferred_element_type=jnp.float32)
    out_ref[...] = out + ob_ref[...]


def _combine(rows, gates, wobf, ob):
    grid = (_N // _TT,)
    return pl.pallas_call(
        _combine_body,
        grid=grid,
        in_specs=[
            pl.BlockSpec((_TT, _D), lambda i: (i, 0)),
            pl.BlockSpec((_TT, _D), lambda i: (i + _N // _TT, 0)),
            pl.BlockSpec((_TT, _K), lambda i: (i, 0)),
            pl.BlockSpec((_D, _D), lambda i: (0, 0)),
            pl.BlockSpec((1, _D), lambda i: (0, 0)),
        ],
        out_specs=pl.BlockSpec((_TT, _D), lambda i: (i, 0)),
        out_shape=jax.ShapeDtypeStruct((_N, _D), jnp.float32),
        compiler_params=pltpu.CompilerParams(
            dimension_semantics=("arbitrary",),
        ),
    )(rows, rows, gates, wobf, ob)


@jax.jit
def kernel(x, router_kernel, router_bias, expert_kernels, expert_biases,
           out_kernel, out_bias):
    b, s, d = x.shape
    xbf = x.reshape(b * s, d).astype(jnp.bfloat16)
    rkbf = router_kernel.astype(jnp.bfloat16)
    ekbf = expert_kernels.astype(jnp.bfloat16)
    wobf = out_kernel.astype(jnp.bfloat16)
    rb = router_bias.reshape(1, _E)
    ob = out_bias.reshape(1, _D)

    dest, gates, tmeta = _route(xbf, rkbf, rb)
    idx = dest.T.reshape(_NW, _NCH, _CH)
    tile_expert = tmeta.reshape(64)[:_G]

    sc_scatter, sc_gather = _sc_kernels()
    xs = sc_scatter(x.reshape(_N, _D), idx)
    ys = _expert_gemm(tile_expert, xs, ekbf, expert_biases.reshape(_E, 1, _D))
    rows = sc_gather(ys, idx)
    out = _combine(rows, gates, wobf, ob)
    return out.reshape(b, s, d)
